# baseline (speedup 1.0000x reference)
import functools

import jax
import jax.numpy as jnp
from jax import lax
from jax.experimental import pallas as pl
from jax.experimental.pallas import tpu as pltpu
from jax.experimental.pallas import tpu_sc as plsc

VOCAB_N = 1000000
EMBED = 64
BATCH = 4096
SEQ = 200

NUM_CORES = 2
NUM_SUBCORES = 16
NUM_WORKERS = NUM_CORES * NUM_SUBCORES
BPW = BATCH // NUM_WORKERS

CHUNKS = ((0, 104), (104, 96))
EV = EMBED // 16
UNROLL = 8
assert SEQ % UNROLL == 0

TLW = 65
CB = 384
NBLK_MAIN = VOCAB_N // CB
LAST_OFF = (NBLK_MAIN - 1) * CB
BLK_PER_W = -(-(NBLK_MAIN) // NUM_WORKERS)
REM = VOCAB_N - NBLK_MAIN * CB
REM_OFF = NBLK_MAIN * CB
CJ = CB // 16


def _leaky(v):
    return jnp.maximum(v, 0.01 * v)


def _sc_linearize_table(tableT, remT):
    mesh = plsc.VectorSubcoreMesh(
        core_axis_name="c", subcore_axis_name="s", num_cores=NUM_CORES
    )

    @functools.partial(
        pl.kernel,
        mesh=mesh,
        compiler_params=pltpu.CompilerParams(needs_layout_passes=False),
        out_type=jax.ShapeDtypeStruct((VOCAB_N * TLW,), jnp.float32),
        scratch_types=[
            pltpu.VMEM((EMBED, CB), jnp.float32),
            pltpu.VMEM((EMBED, CB), jnp.float32),
            pltpu.VMEM((CB * TLW,), jnp.float32),
            pltpu.VMEM((CB * TLW,), jnp.float32),
            pltpu.VMEM((EMBED, REM), jnp.float32),
            pltpu.VMEM((REM * TLW,), jnp.float32),
            pltpu.SemaphoreType.DMA,
            pltpu.SemaphoreType.DMA,
            pltpu.SemaphoreType.DMA,
            pltpu.SemaphoreType.DMA,
        ],
    )
    def k(tT_hbm, remT_hbm, out_hbm, tin0, tin1, tst0, tst1, tinr, tstr,
          sem0, sem1, semo0, semo1):
        wid = lax.axis_index("s") * NUM_CORES + lax.axis_index("c")
        iota_tlw = lax.iota(jnp.int32, 16) * TLW

        def blk_off(k_):
            c = wid + k_ * NUM_WORKERS
            return jnp.minimum(c, NBLK_MAIN - 1) * CB

        def issue(off, tin, sem):
            pltpu.async_copy(tT_hbm.at[:, pl.ds(off, CB)], tin, sem)

        def wait(tin, sem):
            pltpu.make_async_copy(tT_hbm.at[:, pl.ds(0, CB)], tin, sem).wait()

        def transpose_block(tin, tst, width):
            nj = width // 16

            @plsc.parallel_loop(0, EMBED, unroll=4)
            def _(e):
                for j in range(nj):
                    v = _leaky(tin[e, pl.ds(j * 16, 16)])
                    plsc.store_scatter(
                        tst, [iota_tlw + (j * 16 * TLW + e)], v
                    )

        def flush_start(off, tst, semo):
            pltpu.async_copy(
                tst, out_hbm.at[pl.ds(off * TLW, CB * TLW)], semo
            )

        def flush_wait(tst, semo):
            pltpu.make_async_copy(
                tst, out_hbm.at[pl.ds(0, CB * TLW)], semo
            ).wait()

        issue(blk_off(0), tin0, sem0)
        issue(blk_off(1), tin1, sem1)
        wait(tin0, sem0)
        transpose_block(tin0, tst0, CB)
        flush_start(blk_off(0), tst0, semo0)
        issue(blk_off(2), tin0, sem0)
        wait(tin1, sem1)
        transpose_block(tin1, tst1, CB)
        flush_start(blk_off(1), tst1, semo1)

        def step(j, carry):
            b0 = 2 * j
            issue(blk_off(b0 + 1), tin1, sem1)
            wait(tin0, sem0)
            flush_wait(tst0, semo0)
            transpose_block(tin0, tst0, CB)
            flush_start(blk_off(b0), tst0, semo0)
            issue(blk_off(b0 + 2), tin0, sem0)
            wait(tin1, sem1)
            flush_wait(tst1, semo1)
            transpose_block(tin1, tst1, CB)
            flush_start(blk_off(b0 + 1), tst1, semo1)
            return carry

        lax.fori_loop(1, BLK_PER_W // 2, step, 0)
        wait(tin0, sem0)
        flush_wait(tst0, semo0)
        flush_wait(tst1, semo1)

        @pl.when(wid == 0)
        def _():
            pltpu.sync_copy(remT_hbm, tinr)

            @plsc.parallel_loop(0, EMBED, unroll=4)
            def _(e):
                for j in range(REM // 16):
                    v = _leaky(tinr[e, pl.ds(j * 16, 16)])
                    plsc.store_scatter(
                        tstr, [iota_tlw + (j * 16 * TLW + e)], v
                    )

            pltpu.sync_copy(
                tstr, out_hbm.at[pl.ds(REM_OFF * TLW, REM * TLW)]
            )

    return k(tableT, remT)


def _sc_pooled_sums(x, table_lin):
    mesh = plsc.VectorSubcoreMesh(
        core_axis_name="c", subcore_axis_name="s", num_cores=NUM_CORES
    )

    @functools.partial(
        pl.kernel,
        mesh=mesh,
        compiler_params=pltpu.CompilerParams(use_tc_tiling_on_sc=False),
        out_type=jax.ShapeDtypeStruct((BATCH, EMBED), jnp.float32),
        scratch_types=[
            pltpu.VMEM((BPW, SEQ), jnp.int32),
            pltpu.VMEM((SEQ, TLW), jnp.float32),
            pltpu.VMEM((SEQ, TLW), jnp.float32),
            pltpu.VMEM((BPW, EMBED), jnp.float32),
            pltpu.SemaphoreType.DMA,
            pltpu.SemaphoreType.DMA,
        ],
    )
    def k(x_hbm, table_hbm, out_hbm, idx_v, buf0, buf1, pooled, sem0, sem1):
        wid = lax.axis_index("s") * NUM_CORES + lax.axis_index("c")
        base = wid * BPW
        pltpu.sync_copy(x_hbm.at[pl.ds(base, BPW)], idx_v)

        def issue(b, buf, sem):
            for off, ln in CHUNKS:
                pltpu.async_copy(
                    table_hbm.at[idx_v.at[b, pl.ds(off, ln)]],
                    buf.at[pl.ds(off, ln)],
                    sem,
                )

        def wait(buf, sem):
            pltpu.make_async_copy(
                table_hbm.at[idx_v.at[0]], buf, sem
            ).wait()

        def compute(b, buf):
            def body(i, acc):
                accs = list(acc)
                for u in range(UNROLL):
                    s = i * UNROLL + u
                    for e in range(EV):
                        accs[e] = accs[e] + buf[s, pl.ds(e * 16, 16)]
                return tuple(accs)

            zero = jnp.zeros((16,), jnp.float32)
            acc = lax.fori_loop(0, SEQ // UNROLL, body, (zero,) * EV)
            for e in range(EV):
                pooled[b, pl.ds(e * 16, 16)] = acc[e]

        issue(0, buf0, sem0)

        def step(j, carry):
            b = 2 * j
            issue(b + 1, buf1, sem1)
            wait(buf0, sem0)
            compute(b, buf0)
            issue(b + 2, buf0, sem0)
            wait(buf1, sem1)
            compute(b + 1, buf1)
            return carry

        lax.fori_loop(0, BPW // 2 - 1, step, 0)
        issue(BPW - 1, buf1, sem1)
        wait(buf0, sem0)
        compute(BPW - 2, buf0)
        wait(buf1, sem1)
        compute(BPW - 1, buf1)
        pltpu.sync_copy(pooled, out_hbm.at[pl.ds(base, BPW)])

    return k(x, table_lin)


def _tc_mlp(sums, W1, b1, W2, b2):
    def body(s_ref, w1_ref, b1_ref, w2_ref, b2_ref, o_ref):
        pooled = s_ref[...] * (1.0 / SEQ)
        h = lax.dot_general(
            pooled, w1_ref[...], (((1,), (1,)), ((), ())),
            preferred_element_type=jnp.float32,
        ) + b1_ref[...]
        h = jnp.where(h >= 0, h, 0.01 * h)
        logit = jnp.sum(h * w2_ref[...], axis=1, keepdims=True) + b2_ref[0, 0]
        o_ref[...] = jax.nn.sigmoid(logit)

    out = pl.pallas_call(
        body,
        out_shape=jax.ShapeDtypeStruct((BATCH, 1), jnp.float32),
    )(sums, W1, b1, W2, b2.reshape(1, 1))
    return jnp.squeeze(out, -1)


def kernel(x, table, W1, b1, W2, b2):
    x = x.astype(jnp.int32)
    tableT = jnp.swapaxes(table, 0, 1)
    remT = lax.slice(tableT, (0, REM_OFF), (EMBED, VOCAB_N))
    lin = _sc_linearize_table(tableT, remT)
    table_lin = lin.reshape(VOCAB_N, TLW)
    sums = _sc_pooled_sums(x, table_lin)
    return _tc_mlp(sums, W1, b1, W2, b2)


# speedup vs baseline: 2.5733x; 2.5733x over previous
import functools

import jax
import jax.numpy as jnp
from jax import lax
from jax.experimental import pallas as pl
from jax.experimental.pallas import tpu as pltpu
from jax.experimental.pallas import tpu_sc as plsc

VOCAB_N = 1000000
EMBED = 64
BATCH = 4096
SEQ = 200

NUM_CORES = 2
NUM_SUBCORES = 16
NUM_WORKERS = NUM_CORES * NUM_SUBCORES
BPW = BATCH // NUM_WORKERS

CHUNKS = ((0, 104), (104, 96))
EV = EMBED // 16
UNROLL = 8
assert SEQ % UNROLL == 0

RB = 320
NBLK = VOCAB_N // RB
BLK_PER_W = -(-NBLK // NUM_WORKERS)


def _leaky(v):
    return jnp.maximum(v, 0.01 * v)


def _sc_depad_table(table):
    mesh = plsc.VectorSubcoreMesh(
        core_axis_name="c", subcore_axis_name="s", num_cores=NUM_CORES
    )

    @functools.partial(
        pl.kernel,
        mesh=mesh,
        compiler_params=pltpu.CompilerParams(needs_layout_passes=False),
        out_type=jax.ShapeDtypeStruct((VOCAB_N * EMBED,), jnp.float32),
        scratch_types=[
            pltpu.VMEM((RB, EMBED), jnp.float32),
            pltpu.VMEM((RB, EMBED), jnp.float32),
            pltpu.VMEM((RB * EMBED,), jnp.float32),
            pltpu.VMEM((RB * EMBED,), jnp.float32),
            pltpu.SemaphoreType.DMA,
            pltpu.SemaphoreType.DMA,
            pltpu.SemaphoreType.DMA,
            pltpu.SemaphoreType.DMA,
        ],
    )
    def k(t_hbm, out_hbm, tin0, tin1, tst0, tst1, sem0, sem1, semo0, semo1):
        wid = lax.axis_index("s") * NUM_CORES + lax.axis_index("c")

        def blk_off(k_):
            c = wid + k_ * NUM_WORKERS
            return jnp.minimum(c, NBLK - 1) * RB

        def issue(off, tin, sem):
            pltpu.async_copy(t_hbm.at[pl.ds(off, RB)], tin, sem)

        def wait(tin, sem):
            pltpu.make_async_copy(t_hbm.at[pl.ds(0, RB)], tin, sem).wait()

        def depad_block(tin, tst):
            @plsc.parallel_loop(0, RB, unroll=2)
            def _(r):
                for eg in range(EV):
                    v = _leaky(tin[r, pl.ds(eg * 16, 16)])
                    tst[pl.ds(r * EMBED + eg * 16, 16)] = v

        def flush_start(off, tst, semo):
            pltpu.async_copy(
                tst, out_hbm.at[pl.ds(off * EMBED, RB * EMBED)], semo
            )

        def flush_wait(tst, semo):
            pltpu.make_async_copy(
                tst, out_hbm.at[pl.ds(0, RB * EMBED)], semo
            ).wait()

        issue(blk_off(0), tin0, sem0)
        issue(blk_off(1), tin1, sem1)
        wait(tin0, sem0)
        depad_block(tin0, tst0)
        flush_start(blk_off(0), tst0, semo0)
        issue(blk_off(2), tin0, sem0)
        wait(tin1, sem1)
        depad_block(tin1, tst1)
        flush_start(blk_off(1), tst1, semo1)

        def step(j, carry):
            b0 = 2 * j
            issue(blk_off(b0 + 1), tin1, sem1)
            wait(tin0, sem0)
            flush_wait(tst0, semo0)
            depad_block(tin0, tst0)
            flush_start(blk_off(b0), tst0, semo0)
            issue(blk_off(b0 + 2), tin0, sem0)
            wait(tin1, sem1)
            flush_wait(tst1, semo1)
            depad_block(tin1, tst1)
            flush_start(blk_off(b0 + 1), tst1, semo1)
            return carry

        lax.fori_loop(1, BLK_PER_W // 2, step, 0)
        wait(tin0, sem0)
        flush_wait(tst0, semo0)
        flush_wait(tst1, semo1)

    return k(table)


def _sc_pooled_sums(x, table_lin):
    mesh = plsc.VectorSubcoreMesh(
        core_axis_name="c", subcore_axis_name="s", num_cores=NUM_CORES
    )

    @functools.partial(
        pl.kernel,
        mesh=mesh,
        compiler_params=pltpu.CompilerParams(use_tc_tiling_on_sc=False),
        out_type=jax.ShapeDtypeStruct((BATCH, EMBED), jnp.float32),
        scratch_types=[
            pltpu.VMEM((BPW, SEQ), jnp.int32),
            pltpu.VMEM((SEQ, EMBED), jnp.float32),
            pltpu.VMEM((SEQ, EMBED), jnp.float32),
            pltpu.VMEM((BPW, EMBED), jnp.float32),
            pltpu.SemaphoreType.DMA,
            pltpu.SemaphoreType.DMA,
        ],
    )
    def k(x_hbm, table_hbm, out_hbm, idx_v, buf0, buf1, pooled, sem0, sem1):
        wid = lax.axis_index("s") * NUM_CORES + lax.axis_index("c")
        base = wid * BPW
        pltpu.sync_copy(x_hbm.at[pl.ds(base, BPW)], idx_v)

        def issue(b, buf, sem):
            for off, ln in CHUNKS:
                pltpu.async_copy(
                    table_hbm.at[idx_v.at[b, pl.ds(off, ln)]],
                    buf.at[pl.ds(off, ln)],
                    sem,
                )

        def wait(buf, sem):
            pltpu.make_async_copy(
                table_hbm.at[idx_v.at[0]], buf, sem
            ).wait()

        def compute(b, buf):
            def body(i, acc):
                accs = list(acc)
                for u in range(UNROLL):
                    s = i * UNROLL + u
                    for e in range(EV):
                        accs[e] = accs[e] + buf[s, pl.ds(e * 16, 16)]
                return tuple(accs)

            zero = jnp.zeros((16,), jnp.float32)
            acc = lax.fori_loop(0, SEQ // UNROLL, body, (zero,) * EV)
            for e in range(EV):
                pooled[b, pl.ds(e * 16, 16)] = acc[e]

        issue(0, buf0, sem0)

        def step(j, carry):
            b = 2 * j
            issue(b + 1, buf1, sem1)
            wait(buf0, sem0)
            compute(b, buf0)
            issue(b + 2, buf0, sem0)
            wait(buf1, sem1)
            compute(b + 1, buf1)
            return carry

        lax.fori_loop(0, BPW // 2 - 1, step, 0)
        issue(BPW - 1, buf1, sem1)
        wait(buf0, sem0)
        compute(BPW - 2, buf0)
        wait(buf1, sem1)
        compute(BPW - 1, buf1)
        pltpu.sync_copy(pooled, out_hbm.at[pl.ds(base, BPW)])

    return k(x, table_lin)


def _tc_mlp(sums, W1, b1, W2, b2):
    def body(s_ref, w1_ref, b1_ref, w2_ref, b2_ref, o_ref):
        pooled = s_ref[...] * (1.0 / SEQ)
        h = lax.dot_general(
            pooled, w1_ref[...], (((1,), (1,)), ((), ())),
            preferred_element_type=jnp.float32,
        ) + b1_ref[...]
        h = jnp.where(h >= 0, h, 0.01 * h)
        logit = jnp.sum(h * w2_ref[...], axis=1, keepdims=True) + b2_ref[0, 0]
        o_ref[...] = jax.nn.sigmoid(logit)

    out = pl.pallas_call(
        body,
        out_shape=jax.ShapeDtypeStruct((BATCH, 1), jnp.float32),
    )(sums, W1, b1, W2, b2.reshape(1, 1))
    return jnp.squeeze(out, -1)


def kernel(x, table, W1, b1, W2, b2):
    x = x.astype(jnp.int32)
    lin = _sc_depad_table(table)
    table_lin = lin.reshape(VOCAB_N, EMBED)
    sums = _sc_pooled_sums(x, table_lin)
    return _tc_mlp(sums, W1, b1, W2, b2)


# speedup vs baseline: 2.6063x; 1.0128x over previous
import functools

import jax
import jax.numpy as jnp
from jax import lax
from jax.experimental import pallas as pl
from jax.experimental.pallas import tpu as pltpu
from jax.experimental.pallas import tpu_sc as plsc

EMBED = 64
BATCH = 4096
SEQ = 200

NUM_CORES = 2
NUM_SUBCORES = 16
NUM_WORKERS = NUM_CORES * NUM_SUBCORES
BPW = BATCH // NUM_WORKERS

CHUNKS = ((0, 104), (104, 96))
TW = 128
EV = EMBED // 16
UNROLL = 8
assert SEQ % UNROLL == 0


def _sc_pooled_sums(x, table):
    mesh = plsc.VectorSubcoreMesh(
        core_axis_name="c", subcore_axis_name="s", num_cores=NUM_CORES
    )

    @functools.partial(
        pl.kernel,
        mesh=mesh,
        compiler_params=pltpu.CompilerParams(use_tc_tiling_on_sc=False),
        out_type=jax.ShapeDtypeStruct((BATCH, EMBED), jnp.float32),
        scratch_types=[
            pltpu.VMEM((BPW, SEQ), jnp.int32),
            pltpu.VMEM((SEQ, EMBED), jnp.float32),
            pltpu.VMEM((SEQ, EMBED), jnp.float32),
            pltpu.VMEM((BPW, EMBED), jnp.float32),
            pltpu.SemaphoreType.DMA,
            pltpu.SemaphoreType.DMA,
        ],
    )
    def k(x_hbm, table_hbm, out_hbm, idx_v, buf0, buf1, pooled, sem0, sem1):
        wid = lax.axis_index("s") * NUM_CORES + lax.axis_index("c")
        base = wid * BPW
        pltpu.sync_copy(x_hbm.at[pl.ds(base, BPW)], idx_v)

        def issue(b, buf, sem):
            for off, ln in CHUNKS:
                pltpu.async_copy(
                    table_hbm.at[idx_v.at[b, pl.ds(off, ln)]],
                    buf.at[pl.ds(off, ln)],
                    sem,
                )

        def wait(buf, sem):
            pltpu.make_async_copy(
                table_hbm.at[idx_v.at[0]], buf, sem
            ).wait()

        def compute(b, buf):
            def body(i, acc):
                accs = list(acc)
                for u in range(UNROLL):
                    s = i * UNROLL + u
                    for e in range(EV):
                        v = buf[s, pl.ds(e * 16, 16)]
                        accs[e] = accs[e] + jnp.maximum(v, 0.01 * v)
                return tuple(accs)

            zero = jnp.zeros((16,), jnp.float32)
            acc = lax.fori_loop(0, SEQ // UNROLL, body, (zero,) * EV)
            for e in range(EV):
                pooled[b, pl.ds(e * 16, 16)] = acc[e]

        issue(0, buf0, sem0)

        def step(j, carry):
            b = 2 * j
            issue(b + 1, buf1, sem1)
            wait(buf0, sem0)
            compute(b, buf0)
            issue(b + 2, buf0, sem0)
            wait(buf1, sem1)
            compute(b + 1, buf1)
            return carry

        lax.fori_loop(0, BPW // 2 - 1, step, 0)
        issue(BPW - 1, buf1, sem1)
        wait(buf0, sem0)
        compute(BPW - 2, buf0)
        wait(buf1, sem1)
        compute(BPW - 1, buf1)
        pltpu.sync_copy(pooled, out_hbm.at[pl.ds(base, BPW)])

    return k(x, table)


def _tc_mlp(sums, W1, b1, W2, b2):
    def body(s_ref, w1_ref, b1_ref, w2_ref, b2_ref, o_ref):
        pooled = s_ref[...] * (1.0 / SEQ)
        h = lax.dot_general(
            pooled, w1_ref[...], (((1,), (1,)), ((), ())),
            preferred_element_type=jnp.float32,
        ) + b1_ref[...]
        h = jnp.where(h >= 0, h, 0.01 * h)
        logit = jnp.sum(h * w2_ref[...], axis=1, keepdims=True) + b2_ref[0, 0]
        o_ref[...] = jax.nn.sigmoid(logit)

    out = pl.pallas_call(
        body,
        out_shape=jax.ShapeDtypeStruct((BATCH, 1), jnp.float32),
    )(sums, W1, b1, W2, b2.reshape(1, 1))
    return jnp.squeeze(out, -1)


def kernel(x, table, W1, b1, W2, b2):
    x = x.astype(jnp.int32)
    sums = _sc_pooled_sums(x, table)
    return _tc_mlp(sums, W1, b1, W2, b2)
